# baseline (speedup 1.0000x reference)
import functools

import jax
import jax.numpy as jnp
from jax import lax
from jax.experimental import pallas as pl
from jax.experimental.pallas import tpu as pltpu
from jax.experimental.pallas import tpu_sc as plsc

_K = 1024
_D = 64
_N = 16384
_T = 2048
_G = _N // _T
_CC = 0.25

_NW = 32
_BW = _N // _NW


def _vq_kernel(x_ref, emb_ref, embT_ref,
               loss_ref, perp_ref, enc_ref, idx_ref,
               sse_ref, cnt_ref, embn2T_ref, e2_ref):
    i = pl.program_id(0)

    @pl.when(i == 0)
    def _init():
        sse_ref[...] = jnp.zeros_like(sse_ref)
        cnt_ref[...] = jnp.zeros_like(cnt_ref)
        embT = embT_ref[...]
        embn2T_ref[...] = embT * (-2.0)
        e2_ref[...] = jnp.sum(embT * embT, axis=0, keepdims=True)

    x = x_ref[...]

    x2 = jnp.sum(x * x, axis=1, keepdims=True)
    mm = jax.lax.dot_general(x, embn2T_ref[...], (((1,), (0,)), ((), ())),
                             preferred_element_type=jnp.float32)
    dist = (x2 + e2_ref[...]) + mm

    minv = jnp.min(dist, axis=1, keepdims=True)
    iota_k = jax.lax.broadcasted_iota(jnp.int32, (_T, _K), 1)
    idx = jnp.min(jnp.where(dist == minv, iota_k, _K),
                  axis=1, keepdims=True)

    enc = (iota_k == idx).astype(jnp.float32)
    enc_ref[...] = enc

    idx_ref[...] = jnp.transpose(idx)

    sse_ref[...] += minv
    cnt_ref[...] += jnp.sum(enc, axis=0, keepdims=True)

    @pl.when(i == _G - 1)
    def _finish():
        sse = jnp.sum(sse_ref[...], keepdims=True)
        loss_ref[...] = sse * ((1.0 + _CC) / (_N * _D))
        p = cnt_ref[...] * (1.0 / _N)
        ent = jnp.sum(p * jnp.log(p + 1e-10), axis=(0, 1), keepdims=True)
        perp_ref[...] = jnp.exp(-ent)


@functools.partial(
    pl.kernel,
    mesh=plsc.VectorSubcoreMesh(core_axis_name="c", subcore_axis_name="s"),
    out_type=jax.ShapeDtypeStruct((_N, 128), jnp.float32),
    scratch_types=[
        pltpu.VMEM((_BW // 128, 128), jnp.int32),
        pltpu.VMEM((_BW, 128), jnp.float32),
        pltpu.SemaphoreType.DMA,
    ],
)
def _sc_gather(emb_hbm, idx_hbm, out_hbm, idx_v, rows_v, sem):
    wid = lax.axis_index("s") * 2 + lax.axis_index("c")
    base = wid * _BW
    for j in range(_BW // 128):
        pltpu.sync_copy(idx_hbm.at[pl.ds(base + j * 128, 128)], idx_v.at[j])
        pltpu.async_copy(emb_hbm.at[idx_v.at[j]],
                         rows_v.at[pl.ds(j * 128, 128)], sem).wait()
    pltpu.sync_copy(rows_v, out_hbm.at[pl.ds(base, _BW)])


@jax.jit
def kernel(inputs, embedding):
    b, c, h, w = inputs.shape
    flat = jnp.transpose(inputs, (0, 2, 3, 1)).reshape(_N, _D)
    emb_t = embedding.T

    loss2d, perp2d, encodings, idx2d = pl.pallas_call(
        _vq_kernel,
        grid=(_G,),
        in_specs=[
            pl.BlockSpec((_T, _D), lambda i: (i, 0)),
            pl.BlockSpec((_K, _D), lambda i: (0, 0)),
            pl.BlockSpec((_D, _K), lambda i: (0, 0)),
        ],
        out_specs=[
            pl.BlockSpec((1, 1), lambda i: (0, 0)),
            pl.BlockSpec((1, 1), lambda i: (0, 0)),
            pl.BlockSpec((_T, _K), lambda i: (i, 0)),
            pl.BlockSpec((1, _T), lambda i: (0, i)),
        ],
        out_shape=[
            jax.ShapeDtypeStruct((1, 1), jnp.float32),
            jax.ShapeDtypeStruct((1, 1), jnp.float32),
            jax.ShapeDtypeStruct((_N, _K), jnp.float32),
            jax.ShapeDtypeStruct((1, _N), jnp.int32),
        ],
        scratch_shapes=[
            pltpu.VMEM((_T, 1), jnp.float32),
            pltpu.VMEM((1, _K), jnp.float32),
            pltpu.VMEM((_D, _K), jnp.float32),
            pltpu.VMEM((1, _K), jnp.float32),
        ],
    )(flat, embedding, emb_t)

    emb_pad = jnp.concatenate(
        [embedding, jnp.zeros((_K, 128 - _D), jnp.float32)], axis=1)
    q_flat = _sc_gather(emb_pad, idx2d.reshape(_N))[:, :_D]

    quantized = jnp.transpose(q_flat.reshape(b, h, w, c), (0, 3, 1, 2))
    return (loss2d[0, 0], quantized, perp2d[0, 0], encodings)


# speedup vs baseline: 1.6249x; 1.6249x over previous
import jax
import jax.numpy as jnp
from jax.experimental import pallas as pl
from jax.experimental.pallas import tpu as pltpu

_K = 1024
_D = 64
_N = 16384
_T = 2048
_G = _N // _T
_CC = 0.25


def _vq_kernel(x_ref, emb_ref, embT_ref,
               loss_ref, q_ref, perp_ref, enc_ref,
               sse_ref, cnt_ref, embn2T_ref, e2_ref):
    i = pl.program_id(0)

    @pl.when(i == 0)
    def _init():
        sse_ref[...] = jnp.zeros_like(sse_ref)
        cnt_ref[...] = jnp.zeros_like(cnt_ref)
        embT = embT_ref[...]
        embn2T_ref[...] = embT * (-2.0)
        e2_ref[...] = jnp.sum(embT * embT, axis=0, keepdims=True)

    x = x_ref[...]

    x2 = jnp.sum(x * x, axis=1, keepdims=True)
    mm = jax.lax.dot_general(x, embn2T_ref[...], (((1,), (0,)), ((), ())),
                             preferred_element_type=jnp.float32)
    dist = (x2 + e2_ref[...]) + mm

    minv = jnp.min(dist, axis=1, keepdims=True)
    iota_k = jax.lax.broadcasted_iota(jnp.int32, (_T, _K), 1)
    idx = jnp.min(jnp.where(dist == minv, iota_k, _K),
                  axis=1, keepdims=True)

    enc = (iota_k == idx).astype(jnp.float32)
    enc_ref[...] = enc

    q = jax.lax.dot_general(enc, emb_ref[...], (((1,), (0,)), ((), ())),
                            preferred_element_type=jnp.float32)
    q_ref[...] = q

    sse_ref[...] += minv
    cnt_ref[...] += jnp.sum(enc, axis=0, keepdims=True)

    @pl.when(i == _G - 1)
    def _finish():
        sse = jnp.sum(sse_ref[...], keepdims=True)
        loss_ref[...] = sse * ((1.0 + _CC) / (_N * _D))
        p = cnt_ref[...] * (1.0 / _N)
        ent = jnp.sum(p * jnp.log(p + 1e-10), axis=(0, 1), keepdims=True)
        perp_ref[...] = jnp.exp(-ent)


@jax.jit
def kernel(inputs, embedding):
    b, c, h, w = inputs.shape
    flat = jnp.transpose(inputs, (0, 2, 3, 1)).reshape(_N, _D)
    emb_t = embedding.T

    loss2d, q_flat, perp2d, encodings = pl.pallas_call(
        _vq_kernel,
        grid=(_G,),
        in_specs=[
            pl.BlockSpec((_T, _D), lambda i: (i, 0)),
            pl.BlockSpec((_K, _D), lambda i: (0, 0)),
            pl.BlockSpec((_D, _K), lambda i: (0, 0)),
        ],
        out_specs=[
            pl.BlockSpec((1, 1), lambda i: (0, 0)),
            pl.BlockSpec((_T, _D), lambda i: (i, 0)),
            pl.BlockSpec((1, 1), lambda i: (0, 0)),
            pl.BlockSpec((_T, _K), lambda i: (i, 0)),
        ],
        out_shape=[
            jax.ShapeDtypeStruct((1, 1), jnp.float32),
            jax.ShapeDtypeStruct((_N, _D), jnp.float32),
            jax.ShapeDtypeStruct((1, 1), jnp.float32),
            jax.ShapeDtypeStruct((_N, _K), jnp.float32),
        ],
        scratch_shapes=[
            pltpu.VMEM((_T, 1), jnp.float32),
            pltpu.VMEM((1, _K), jnp.float32),
            pltpu.VMEM((_D, _K), jnp.float32),
            pltpu.VMEM((1, _K), jnp.float32),
        ],
    )(flat, embedding, emb_t)

    quantized = jnp.transpose(q_flat.reshape(b, h, w, c), (0, 3, 1, 2))
    return (loss2d[0, 0], quantized, perp2d[0, 0], encodings)
